# Initial kernel scaffold; baseline (speedup 1.0000x reference)
import jax
import jax.numpy as jnp
from jax.experimental import pallas as pl


def kernel(x, fast_state, weight, bias, alpha, mod_w1, mod_b1, mod_w2, mod_b2, eta_base, decay):
    raise NotImplementedError("write your pallas kernel here")


# speedup vs baseline: 1.2350x; 1.2350x over previous
import jax
import jax.numpy as jnp
from jax import lax
from jax.experimental import pallas as pl
from jax.experimental.pallas import tpu as pltpu

B = 128
IN_F = 1024
OUT_F = 1024
HID = 512

_DN = (((1,), (1,)), ((), ()))


def _body(s_ref, x_ref, w_ref, b_ref, a_ref, w1_ref, b1_ref, w2_ref, b2_ref,
          fs_ref, y_ref, nfs_ref):
    b = pl.program_id(0)
    xb = x_ref[pl.ds(b, 1), :]
    fs = fs_ref[0]

    y_base = lax.dot_general(xb, w_ref[...], _DN,
                             preferred_element_type=jnp.float32)
    fast_contrib = lax.dot_general(xb, fs, _DN,
                                   preferred_element_type=jnp.float32)

    h = jnp.maximum(
        lax.dot_general(xb, w1_ref[...], _DN,
                        preferred_element_type=jnp.float32) + b1_ref[...],
        0.0)
    ph = lax.dot_general(h, w2_ref[...], _DN,
                         preferred_element_type=jnp.float32) + b2_ref[...]
    phiv = jax.nn.sigmoid(ph)
    phi = pltpu.repeat(phiv, OUT_F // 128, axis=1)

    y = y_base + b_ref[...] + fast_contrib * (a_ref[...] * phi)
    y_ref[0] = y

    y_scaled = y * (phi * s_ref[0])
    outer = lax.dot_general(y_scaled, xb, (((0,), (0,)), ((), ())),
                            preferred_element_type=jnp.float32)
    nfs_ref[0] = fs * s_ref[1] + outer


def kernel(x, fast_state, weight, bias, alpha, mod_w1, mod_b1, mod_w2, mod_b2,
           eta_base, decay):
    scalars = jnp.stack([jnp.asarray(eta_base, jnp.float32),
                         jnp.asarray(decay, jnp.float32)])
    bias2 = bias.reshape(1, OUT_F)
    alpha2 = alpha.reshape(1, OUT_F)
    mod_b1_2 = mod_b1.reshape(1, HID)
    w2_rep = jnp.broadcast_to(mod_w2.reshape(1, HID), (128, HID))
    b2_row = jnp.broadcast_to(mod_b2.reshape(1, 1), (1, 128))

    y3, new_fast = pl.pallas_call(
        _body,
        grid=(B,),
        in_specs=[
            pl.BlockSpec(memory_space=pltpu.SMEM),
            pl.BlockSpec((B, IN_F), lambda b: (0, 0)),
            pl.BlockSpec((OUT_F, IN_F), lambda b: (0, 0)),
            pl.BlockSpec((1, OUT_F), lambda b: (0, 0)),
            pl.BlockSpec((1, OUT_F), lambda b: (0, 0)),
            pl.BlockSpec((HID, IN_F), lambda b: (0, 0)),
            pl.BlockSpec((1, HID), lambda b: (0, 0)),
            pl.BlockSpec((128, HID), lambda b: (0, 0)),
            pl.BlockSpec((1, 128), lambda b: (0, 0)),
            pl.BlockSpec((1, OUT_F, IN_F), lambda b: (b, 0, 0)),
        ],
        out_specs=[
            pl.BlockSpec((1, 1, OUT_F), lambda b: (b, 0, 0)),
            pl.BlockSpec((1, OUT_F, IN_F), lambda b: (b, 0, 0)),
        ],
        out_shape=[
            jax.ShapeDtypeStruct((B, 1, OUT_F), jnp.float32),
            jax.ShapeDtypeStruct((B, OUT_F, IN_F), jnp.float32),
        ],
        compiler_params=pltpu.CompilerParams(
            dimension_semantics=("arbitrary",),
            vmem_limit_bytes=48 * 1024 * 1024,
        ),
        name="plastic_linear",
    )(scalars, x, weight, bias2, alpha2, mod_w1, mod_b1_2, w2_rep, b2_row,
      fast_state)

    return (y3.reshape(B, OUT_F), new_fast)


# speedup vs baseline: 1.4245x; 1.1534x over previous
import jax
import jax.numpy as jnp
from jax import lax
from jax.experimental import pallas as pl
from jax.experimental.pallas import tpu as pltpu

B = 128
IN_F = 1024
OUT_F = 1024
HID = 512
ROWS = 2

_DN = (((1,), (1,)), ((), ()))


def _prologue(x_ref, w_ref, b_ref, w1_ref, b1_ref, w2_ref, b2_ref,
              ybb_ref, phi_ref):
    x = x_ref[...]
    ybb_ref[...] = lax.dot_general(
        x, w_ref[...], _DN, preferred_element_type=jnp.float32) + b_ref[...]
    h = jnp.maximum(
        lax.dot_general(x, w1_ref[...], _DN,
                        preferred_element_type=jnp.float32) + b1_ref[...],
        0.0)
    ph = lax.dot_general(h, w2_ref[...], _DN,
                         preferred_element_type=jnp.float32) + b2_ref[...]
    phi_ref[...] = jax.nn.sigmoid(ph)


def _main(s_ref, x_ref, ybb_ref, a_ref, phi_ref, fs_ref, y_ref, nfs_ref):
    g = pl.program_id(0)
    for r in range(ROWS):
        b = g * ROWS + r
        xb = x_ref[pl.ds(b, 1), :]
        fs = fs_ref[r]
        fc = lax.dot_general(xb, fs, _DN,
                             preferred_element_type=jnp.float32)
        phi = pltpu.repeat(phi_ref[pl.ds(b, 1), :], OUT_F // 128, axis=1)
        y = ybb_ref[pl.ds(b, 1), :] + fc * (a_ref[...] * phi)
        y_ref[r] = y
        y_scaled = y * (phi * s_ref[0])
        outer = lax.dot_general(y_scaled, xb, (((0,), (0,)), ((), ())),
                                preferred_element_type=jnp.float32)
        nfs_ref[r] = fs * s_ref[1] + outer


def kernel(x, fast_state, weight, bias, alpha, mod_w1, mod_b1, mod_w2, mod_b2,
           eta_base, decay):
    scalars = jnp.stack([jnp.asarray(eta_base, jnp.float32),
                         jnp.asarray(decay, jnp.float32)])
    bias2 = bias.reshape(1, OUT_F)
    alpha2 = alpha.reshape(1, OUT_F)
    mod_b1_2 = mod_b1.reshape(1, HID)
    w2_rep = jnp.broadcast_to(mod_w2.reshape(1, HID), (128, HID))
    b2_row = jnp.broadcast_to(mod_b2.reshape(1, 1), (1, 128))

    ybb, phi = pl.pallas_call(
        _prologue,
        out_shape=[
            jax.ShapeDtypeStruct((B, OUT_F), jnp.float32),
            jax.ShapeDtypeStruct((B, 128), jnp.float32),
        ],
        name="plastic_linear_prologue",
    )(x, weight, bias2, mod_w1, mod_b1_2, w2_rep, b2_row)

    y3, new_fast = pl.pallas_call(
        _main,
        grid=(B // ROWS,),
        in_specs=[
            pl.BlockSpec(memory_space=pltpu.SMEM),
            pl.BlockSpec((B, IN_F), lambda g: (0, 0)),
            pl.BlockSpec((B, OUT_F), lambda g: (0, 0)),
            pl.BlockSpec((1, OUT_F), lambda g: (0, 0)),
            pl.BlockSpec((B, 128), lambda g: (0, 0)),
            pl.BlockSpec((ROWS, OUT_F, IN_F), lambda g: (g, 0, 0)),
        ],
        out_specs=[
            pl.BlockSpec((ROWS, 1, OUT_F), lambda g: (g, 0, 0)),
            pl.BlockSpec((ROWS, OUT_F, IN_F), lambda g: (g, 0, 0)),
        ],
        out_shape=[
            jax.ShapeDtypeStruct((B, 1, OUT_F), jnp.float32),
            jax.ShapeDtypeStruct((B, OUT_F, IN_F), jnp.float32),
        ],
        compiler_params=pltpu.CompilerParams(
            dimension_semantics=("arbitrary",),
            vmem_limit_bytes=48 * 1024 * 1024,
        ),
        name="plastic_linear_stream",
    )(scalars, x, ybb, alpha2, phi, fast_state)

    return (y3.reshape(B, OUT_F), new_fast)
